# baseline (speedup 1.0000x reference)
import functools

import jax
import jax.numpy as jnp
from jax import lax
from jax.experimental import pallas as pl
from jax.experimental.pallas import tpu as pltpu
from jax.experimental.pallas import tpu_sc as plsc

_BN = 8192


def _argmin_body(x_ref, n_ref, w_ref, out_ref, bv_ref, bi_ref, *, n_total):
    b = pl.program_id(0)
    j = pl.program_id(1)
    nb = pl.num_programs(1)

    @pl.when(j == 0)
    def _():
        bv_ref[...] = jnp.full_like(bv_ref[...], jnp.inf)
        bi_ref[...] = jnp.zeros_like(bi_ref[...])

    x = x_ref[0]
    w = w_ref[...]
    mean = lax.dot_general(x, w, (((1,), (1,)), ((), ())),
                           preferred_element_type=jnp.float32)
    mean_t = lax.transpose(mean, (1, 0))
    noise = n_ref[:, 0, 0, :]
    samples = noise + mean_t

    col = jax.lax.broadcasted_iota(jnp.int32, samples.shape, 1)
    valid = (j * _BN + col) < n_total
    samples = jnp.where(valid, samples, jnp.inf)

    local_min = jnp.min(samples, axis=1, keepdims=True)
    local_arg = jnp.argmin(samples, axis=1).astype(jnp.int32)
    local_arg = local_arg.reshape(samples.shape[0], 1)

    better = local_min < bv_ref[...]
    gidx = b * n_total + j * _BN + local_arg
    bv_ref[...] = jnp.where(better, local_min, bv_ref[...])
    bi_ref[...] = jnp.where(better, gidx, bi_ref[...])

    @pl.when(j == nb - 1)
    def _():
        out_ref[0] = bi_ref[...]


def _tc_argmin(X, noise, w2, interpret=False):
    B, N, d = X.shape
    S = noise.shape[0]
    nb = (N + _BN - 1) // _BN
    noise4 = noise.reshape(S, B, 1, N)
    grid = (B, nb)
    return pl.pallas_call(
        functools.partial(_argmin_body, n_total=N),
        grid=grid,
        in_specs=[
            pl.BlockSpec((1, _BN, d), lambda b, j: (b, j, 0)),
            pl.BlockSpec((S, 1, 1, _BN), lambda b, j: (0, b, 0, j)),
            pl.BlockSpec((1, d), lambda b, j: (0, 0)),
        ],
        out_specs=pl.BlockSpec((1, S, 1), lambda b, j: (b, 0, 0)),
        out_shape=jax.ShapeDtypeStruct((B, S, 1), jnp.int32),
        scratch_shapes=[
            pltpu.VMEM((S, 1), jnp.float32),
            pltpu.VMEM((S, 1), jnp.int32),
        ],
        interpret=interpret,
    )(X, noise4, w2)


def _sc_gather(table128, idx_padded, d):
    L = 16
    n_workers = 16
    rows_pw = 8

    mesh = plsc.VectorSubcoreMesh(core_axis_name="c", subcore_axis_name="s")

    @functools.partial(
        pl.kernel,
        mesh=mesh,
        out_type=jax.ShapeDtypeStruct((n_workers * rows_pw * d,), jnp.float32),
        scratch_types=[
            pltpu.VMEM((L,), jnp.int32),
            pltpu.VMEM((L,), jnp.int32),
            pltpu.VMEM((L,), jnp.int32),
            pltpu.VMEM((L, 128), jnp.float32),
            pltpu.VMEM((rows_pw * d,), jnp.float32),
            pltpu.SemaphoreType.DMA,
        ],
    )
    def gather_kernel(table_hbm, idx_hbm, out_hbm,
                      idx_v, q_v, m_v, rows_v, out_v, sem):
        cid = lax.axis_index("c")
        sid = lax.axis_index("s")
        wid = sid * 2 + cid

        @pl.when(wid < n_workers)
        def _():
            pltpu.sync_copy(idx_hbm.at[wid], idx_v)
            iv = idx_v[...]
            q_v[...] = lax.shift_right_logical(iv, 2)
            m_v[...] = iv & 3
            pltpu.async_copy(table_hbm.at[q_v], rows_v, sem).wait()
            mv_all = m_v[...]
            for r in range(rows_pw):
                mvec = jnp.full((L,), mv_all[r], jnp.int32)
                for h in range(d // L):
                    vals = jnp.zeros((L,), jnp.float32)
                    for k in range(4):
                        sk = rows_v[r, pl.ds(k * d + h * L, L)]
                        wk = (1 - jnp.minimum(jnp.abs(mvec - k), 1)
                              ).astype(jnp.float32)
                        vals = vals + sk * wk
                    out_v[pl.ds((r * d // L + h) * L, L)] = vals
            pltpu.sync_copy(out_v, out_hbm.at[pl.ds(wid * rows_pw * d,
                                                    rows_pw * d)])

    return gather_kernel(table128, idx_padded)


def kernel(X, noise, W, num_samples):
    B, N, d = X.shape
    S = noise.shape[0]
    w2 = W.reshape(1, d)
    idcs = _tc_argmin(X, noise, w2)
    idx_padded = jnp.zeros((B, 16), jnp.int32).at[:, :S].set(
        idcs.reshape(B, S))
    table128 = X.reshape(B * N * d // 128, 128)
    flat = _sc_gather(table128, idx_padded, d)
    return flat.reshape(B, S, d)


# speedup vs baseline: 7.9718x; 7.9718x over previous
import functools

import jax
import jax.numpy as jnp
from jax import lax
from jax.experimental import pallas as pl
from jax.experimental.pallas import tpu as pltpu
from jax.experimental.pallas import tpu_sc as plsc

_BN = 16384


def _argmin_body(x_ref, n_ref, w_ref, out_ref, bv_ref, bi_ref, *, n_total):
    j = pl.program_id(1)
    nb = pl.num_programs(1)

    @pl.when(j == 0)
    def _():
        bv_ref[...] = jnp.full_like(bv_ref[...], jnp.inf)
        bi_ref[...] = jnp.zeros_like(bi_ref[...])

    x = x_ref[0]
    w8 = w_ref[...]
    noise = n_ref[:, 0, 0, :]
    samples = lax.dot_general(w8, x, (((1,), (0,)), ((), ())),
                              preferred_element_type=jnp.float32) + noise

    col = jax.lax.broadcasted_iota(jnp.int32, samples.shape, 1)
    valid = (j * _BN + col) < n_total
    samples = jnp.where(valid, samples, jnp.inf)

    local_min = jnp.min(samples, axis=1, keepdims=True)
    local_arg = jnp.argmin(samples, axis=1).astype(jnp.int32)
    local_arg = local_arg.reshape(samples.shape[0], 1)

    better = local_min < bv_ref[...]
    bv_ref[...] = jnp.where(better, local_min, bv_ref[...])
    bi_ref[...] = jnp.where(better, j * _BN + local_arg, bi_ref[...])

    @pl.when(j == nb - 1)
    def _():
        row = lax.transpose(bi_ref[...], (1, 0))
        out_ref[0] = jnp.concatenate(
            [row, jnp.zeros_like(row)], axis=1)


def _tc_argmin(Xt, noise, w8, interpret=False):
    B, d, N = Xt.shape
    S = noise.shape[0]
    nb = (N + _BN - 1) // _BN
    noise4 = noise.reshape(S, B, 1, N)
    return pl.pallas_call(
        functools.partial(_argmin_body, n_total=N),
        grid=(B, nb),
        in_specs=[
            pl.BlockSpec((1, d, _BN), lambda b, j: (b, 0, j)),
            pl.BlockSpec((S, 1, 1, _BN), lambda b, j: (0, b, 0, j)),
            pl.BlockSpec((S, d), lambda b, j: (0, 0)),
        ],
        out_specs=pl.BlockSpec((1, 1, 2 * S), lambda b, j: (b, 0, 0)),
        out_shape=jax.ShapeDtypeStruct((B, 1, 2 * S), jnp.int32),
        scratch_shapes=[
            pltpu.VMEM((S, 1), jnp.float32),
            pltpu.VMEM((S, 1), jnp.int32),
        ],
        interpret=interpret,
    )(Xt, noise4, w8)


_GATHER_DNUMS = lax.GatherDimensionNumbers(
    offset_dims=(), collapsed_slice_dims=(0,), start_index_map=(0,))


def _lane_bcast(vec, off_vec):
    return lax.gather(vec, off_vec[:, None], _GATHER_DNUMS, slice_sizes=(1,),
                      mode=lax.GatherScatterMode.PROMISE_IN_BOUNDS)


def _sc_gather(Xt, idx16):
    B, d, N = Xt.shape
    S = 8
    L = 16
    mesh = plsc.VectorSubcoreMesh(core_axis_name="c", subcore_axis_name="s")

    @functools.partial(
        pl.kernel,
        mesh=mesh,
        out_type=jax.ShapeDtypeStruct((B, S, d), jnp.float32),
        scratch_types=[
            pltpu.VMEM((2 * S,), jnp.int32),
            pltpu.VMEM((S, d, 128), jnp.float32),
            pltpu.VMEM((S, d), jnp.float32),
            pltpu.SemaphoreType.DMA,
        ],
    )
    def gather_kernel(x_hbm, idx_hbm, out_hbm, idx_v, slabs, col_buf, sem):
        cid = lax.axis_index("c")
        sid = lax.axis_index("s")
        wid = sid * 2 + cid

        iota = lax.iota(jnp.int32, L)

        @pl.when(wid < B)
        def _():
            pltpu.sync_copy(idx_hbm.at[wid, 0], idx_v)
            iv = idx_v[...]
            descs = []
            for s in range(S):
                nb = pl.multiple_of((iv[s] // 128) * 128, 128)
                descs.append(pltpu.async_copy(
                    x_hbm.at[wid, :, pl.ds(nb, 128)], slabs.at[s], sem))
            for dsc in descs:
                dsc.wait()
            for s in range(S):
                c = iv[s] % 128
                ch = c // L
                off = c % L
                off_vec = jnp.full((L,), off, jnp.int32)
                wks = [(1 - jnp.minimum(jnp.abs(ch - k), 1)
                        ).astype(jnp.float32) for k in range(8)]
                for h in range(d // L):
                    acc = jnp.zeros((L,), jnp.float32)
                    for t in range(L):
                        dd = h * L + t
                        sel = jnp.zeros((L,), jnp.float32)
                        for k in range(128 // L):
                            sel = sel + slabs[s, dd, pl.ds(k * L, L)] * wks[k]
                        val = _lane_bcast(sel, off_vec)
                        onehot = (1 - jnp.minimum(jnp.abs(iota - t), 1)
                                  ).astype(jnp.float32)
                        acc = acc + val * onehot
                    col_buf[s, pl.ds(h * L, L)] = acc
            pltpu.sync_copy(col_buf, out_hbm.at[wid])

    return gather_kernel(Xt, idx16)


def kernel(X, noise, W, num_samples):
    B, N, d = X.shape
    S = noise.shape[0]
    Xt = jnp.transpose(X, (0, 2, 1))
    w8 = jnp.broadcast_to(W.reshape(1, d), (S, d))
    idx16 = _tc_argmin(Xt, noise, w8)
    return _sc_gather(Xt, idx16)


# speedup vs baseline: 14.1888x; 1.7799x over previous
import functools

import jax
import jax.numpy as jnp
from jax import lax
from jax.experimental import pallas as pl
from jax.experimental.pallas import tpu as pltpu
from jax.experimental.pallas import tpu_sc as plsc

_BN = 8192


def _argmin_body(x_ref, n_ref, w_ref, out_ref, bv_ref, bi_ref,
                 *, n_total, n_batch):
    j = pl.program_id(0)
    nb = pl.num_programs(0)

    @pl.when(j == 0)
    def _():
        bv_ref[...] = jnp.full_like(bv_ref[...], jnp.inf)
        bi_ref[...] = jnp.zeros_like(bi_ref[...])

    w8 = w_ref[...]
    S = w8.shape[0]
    col = jax.lax.broadcasted_iota(jnp.int32, (S, _BN), 1)
    valid = (j * _BN + col) < n_total

    for b in range(n_batch):
        x = x_ref[b]
        samples = lax.dot_general(w8, x, (((1,), (0,)), ((), ())),
                                  preferred_element_type=jnp.float32)
        samples = samples + n_ref[:, b, :]
        samples = jnp.where(valid, samples, jnp.inf)

        local_min = jnp.min(samples, axis=1, keepdims=True)
        local_arg = jnp.argmin(samples, axis=1).astype(jnp.int32)
        local_arg = local_arg.reshape(S, 1)

        better = local_min < bv_ref[:, b:b + 1]
        bv_ref[:, b:b + 1] = jnp.where(better, local_min, bv_ref[:, b:b + 1])
        bi_ref[:, b:b + 1] = jnp.where(better, j * _BN + local_arg,
                                       bi_ref[:, b:b + 1])

    @pl.when(j == nb - 1)
    def _():
        idx_t = lax.transpose(bi_ref[...], (1, 0))
        out_ref[:, 0, :] = jnp.concatenate(
            [idx_t, jnp.zeros_like(idx_t)], axis=1)


def _tc_argmin(Xt, noise, w8, interpret=False):
    B, d, N = Xt.shape
    S = noise.shape[0]
    nb = (N + _BN - 1) // _BN
    return pl.pallas_call(
        functools.partial(_argmin_body, n_total=N, n_batch=B),
        grid=(nb,),
        in_specs=[
            pl.BlockSpec((B, d, _BN), lambda j: (0, 0, j)),
            pl.BlockSpec((S, B, _BN), lambda j: (0, 0, j)),
            pl.BlockSpec((S, d), lambda j: (0, 0)),
        ],
        out_specs=pl.BlockSpec((B, 1, 2 * S), lambda j: (0, 0, 0)),
        out_shape=jax.ShapeDtypeStruct((B, 1, 2 * S), jnp.int32),
        scratch_shapes=[
            pltpu.VMEM((S, B), jnp.float32),
            pltpu.VMEM((S, B), jnp.int32),
        ],
        interpret=interpret,
    )(Xt, noise, w8)


_GATHER_DNUMS = lax.GatherDimensionNumbers(
    offset_dims=(), collapsed_slice_dims=(0,), start_index_map=(0,))


def _lane_bcast(vec, off_vec):
    return lax.gather(vec, off_vec[:, None], _GATHER_DNUMS, slice_sizes=(1,),
                      mode=lax.GatherScatterMode.PROMISE_IN_BOUNDS)


def _sc_gather(Xt, idx16):
    B, d, N = Xt.shape
    S = 8
    L = 16
    mesh = plsc.VectorSubcoreMesh(core_axis_name="c", subcore_axis_name="s")

    @functools.partial(
        pl.kernel,
        mesh=mesh,
        out_type=jax.ShapeDtypeStruct((B, S, d), jnp.float32),
        scratch_types=[
            pltpu.VMEM((2 * S,), jnp.int32),
            pltpu.VMEM((S, d, 128), jnp.float32),
            pltpu.VMEM((S, d), jnp.float32),
            pltpu.SemaphoreType.DMA,
        ],
    )
    def gather_kernel(x_hbm, idx_hbm, out_hbm, idx_v, slabs, col_buf, sem):
        cid = lax.axis_index("c")
        sid = lax.axis_index("s")
        wid = sid * 2 + cid

        iota = lax.iota(jnp.int32, L)

        @pl.when(wid < B)
        def _():
            pltpu.sync_copy(idx_hbm.at[wid, 0], idx_v)
            iv = idx_v[...]
            descs = []
            for s in range(S):
                nb = pl.multiple_of((iv[s] // 128) * 128, 128)
                descs.append(pltpu.async_copy(
                    x_hbm.at[wid, :, pl.ds(nb, 128)], slabs.at[s], sem))
            for dsc in descs:
                dsc.wait()
            for s in range(S):
                c = iv[s] % 128
                ch = c // L
                off = c % L
                off_vec = jnp.full((L,), off, jnp.int32)
                wks = [(1 - jnp.minimum(jnp.abs(ch - k), 1)
                        ).astype(jnp.float32) for k in range(8)]
                for h in range(d // L):
                    acc = jnp.zeros((L,), jnp.float32)
                    for t in range(L):
                        dd = h * L + t
                        sel = jnp.zeros((L,), jnp.float32)
                        for k in range(128 // L):
                            sel = sel + slabs[s, dd, pl.ds(k * L, L)] * wks[k]
                        val = _lane_bcast(sel, off_vec)
                        onehot = (1 - jnp.minimum(jnp.abs(iota - t), 1)
                                  ).astype(jnp.float32)
                        acc = acc + val * onehot
                    col_buf[s, pl.ds(h * L, L)] = acc
            pltpu.sync_copy(col_buf, out_hbm.at[wid])

    return gather_kernel(Xt, idx16)


def kernel(X, noise, W, num_samples):
    B, N, d = X.shape
    S = noise.shape[0]
    Xt = jnp.transpose(X, (0, 2, 1))
    w8 = jnp.broadcast_to(W.reshape(1, d), (S, d))
    idx16 = _tc_argmin(Xt, noise, w8)
    return _sc_gather(Xt, idx16)


# speedup vs baseline: 14.3372x; 1.0105x over previous
import functools

import jax
import jax.numpy as jnp
from jax import lax
from jax.experimental import pallas as pl
from jax.experimental.pallas import tpu as pltpu
from jax.experimental.pallas import tpu_sc as plsc

_S = 8
_BN = 11264


def _argmin_body(x_ref, n_ref, w_ref, out_ref, bv_ref, bi_ref,
                 *, n_total, n_batch):
    j = pl.program_id(0)
    nb = pl.num_programs(0)

    @pl.when(j == 0)
    def _():
        bv_ref[...] = jnp.full_like(bv_ref[...], jnp.inf)
        bi_ref[...] = jnp.zeros_like(bi_ref[...])

    w8 = jnp.broadcast_to(w_ref[...], (_S, w_ref.shape[1]))
    S = _S
    col = jax.lax.broadcasted_iota(jnp.int32, (S, _BN), 1)
    valid = (j * _BN + col) < n_total

    for b in range(n_batch):
        x = x_ref[b]
        samples = lax.dot_general(w8, x, (((1,), (0,)), ((), ())),
                                  preferred_element_type=jnp.float32)
        samples = samples + n_ref[:, b, :]
        samples = jnp.where(valid, samples, jnp.inf)

        local_min = jnp.min(samples, axis=1, keepdims=True)
        local_arg = jnp.argmin(samples, axis=1).astype(jnp.int32)
        local_arg = local_arg.reshape(S, 1)

        better = local_min < bv_ref[:, b:b + 1]
        bv_ref[:, b:b + 1] = jnp.where(better, local_min, bv_ref[:, b:b + 1])
        bi_ref[:, b:b + 1] = jnp.where(better, j * _BN + local_arg,
                                       bi_ref[:, b:b + 1])

    @pl.when(j == nb - 1)
    def _():
        idx_t = lax.transpose(bi_ref[...], (1, 0))
        out_ref[:, 0, :] = jnp.concatenate(
            [idx_t, jnp.zeros_like(idx_t)], axis=1)


def _tc_argmin(Xt, noise, w8, interpret=False):
    B, d, N = Xt.shape
    S = noise.shape[0]
    nb = (N + _BN - 1) // _BN
    return pl.pallas_call(
        functools.partial(_argmin_body, n_total=N, n_batch=B),
        grid=(nb,),
        in_specs=[
            pl.BlockSpec((B, d, _BN), lambda j: (0, 0, j)),
            pl.BlockSpec((S, B, _BN), lambda j: (0, 0, j)),
            pl.BlockSpec((1, d), lambda j: (0, 0)),
        ],
        out_specs=pl.BlockSpec((B, 1, 2 * S), lambda j: (0, 0, 0)),
        out_shape=jax.ShapeDtypeStruct((B, 1, 2 * S), jnp.int32),
        scratch_shapes=[
            pltpu.VMEM((S, B), jnp.float32),
            pltpu.VMEM((S, B), jnp.int32),
        ],
        interpret=interpret,
    )(Xt, noise, w8)


_GATHER_DNUMS = lax.GatherDimensionNumbers(
    offset_dims=(), collapsed_slice_dims=(0,), start_index_map=(0,))


def _lane_bcast(vec, off_vec):
    return lax.gather(vec, off_vec[:, None], _GATHER_DNUMS, slice_sizes=(1,),
                      mode=lax.GatherScatterMode.PROMISE_IN_BOUNDS)


def _sc_gather(Xt, idx16):
    B, d, N = Xt.shape
    S = 8
    L = 16
    mesh = plsc.VectorSubcoreMesh(core_axis_name="c", subcore_axis_name="s")

    @functools.partial(
        pl.kernel,
        mesh=mesh,
        out_type=jax.ShapeDtypeStruct((B, S, d), jnp.float32),
        scratch_types=[
            pltpu.VMEM((2 * S,), jnp.int32),
            pltpu.VMEM((S, d, 128), jnp.float32),
            pltpu.VMEM((S, d), jnp.float32),
            pltpu.SemaphoreType.DMA,
        ],
    )
    def gather_kernel(x_hbm, idx_hbm, out_hbm, idx_v, slabs, col_buf, sem):
        cid = lax.axis_index("c")
        sid = lax.axis_index("s")
        wid = sid * 2 + cid

        iota = lax.iota(jnp.int32, L)

        @pl.when(wid < B)
        def _():
            pltpu.sync_copy(idx_hbm.at[wid, 0], idx_v)
            iv = idx_v[...]
            descs = []
            for s in range(S):
                nb = pl.multiple_of((iv[s] // 128) * 128, 128)
                descs.append(pltpu.async_copy(
                    x_hbm.at[wid, :, pl.ds(nb, 128)], slabs.at[s], sem))
            for dsc in descs:
                dsc.wait()
            for s in range(S):
                c = iv[s] % 128
                ch = c // L
                off = c % L
                off_vec = jnp.full((L,), off, jnp.int32)
                wks = [(1 - jnp.minimum(jnp.abs(ch - k), 1)
                        ).astype(jnp.float32) for k in range(8)]
                for h in range(d // L):
                    acc = jnp.zeros((L,), jnp.float32)
                    for t in range(L):
                        dd = h * L + t
                        sel = jnp.zeros((L,), jnp.float32)
                        for k in range(128 // L):
                            sel = sel + slabs[s, dd, pl.ds(k * L, L)] * wks[k]
                        val = _lane_bcast(sel, off_vec)
                        onehot = (1 - jnp.minimum(jnp.abs(iota - t), 1)
                                  ).astype(jnp.float32)
                        acc = acc + val * onehot
                    col_buf[s, pl.ds(h * L, L)] = acc
            pltpu.sync_copy(col_buf, out_hbm.at[wid])

    return gather_kernel(Xt, idx16)


def kernel(X, noise, W, num_samples):
    B, N, d = X.shape
    S = noise.shape[0]
    Xt = jnp.transpose(X, (0, 2, 1))
    w2 = W.reshape(1, d)
    idx16 = _tc_argmin(Xt, noise, w2)
    return _sc_gather(Xt, idx16)
